# baseline (speedup 1.0000x reference)
import functools

import jax
import jax.numpy as jnp
from jax import lax
from jax.experimental import pallas as pl
from jax.experimental.pallas import tpu as pltpu
from jax.experimental.pallas import tpu_sc as plsc

L = 16
NC, NS = 2, 16
NW = NC * NS

RI = 32
RO = 2 * RI
NK = 7


def _zero_weave_sc(x3, ztile, *, BC, NCH, H, W):
    Ho, Wo = 2 * H - 1, 2 * W - 1
    ch_per = BC // NW
    NG = ch_per * NK

    mesh = plsc.VectorSubcoreMesh(
        core_axis_name="c", subcore_axis_name="s", num_cores=NC, num_subcores=NS
    )

    @functools.partial(
        pl.kernel,
        out_type=jax.ShapeDtypeStruct((BC // NCH, NCH, Ho, Wo), jnp.float32),
        mesh=mesh,
        scratch_types=[
            pltpu.VMEM((RI, W), jnp.float32),
            pltpu.VMEM((RI, W), jnp.float32),
            pltpu.VMEM((RO, Wo), jnp.float32),
            pltpu.VMEM((RO, Wo), jnp.float32),
            pltpu.SemaphoreType.DMA,
            pltpu.SemaphoreType.DMA,
            pltpu.SemaphoreType.DMA,
            pltpu.SemaphoreType.DMA,
        ],
        compiler_params=pltpu.CompilerParams(
            use_tc_tiling_on_sc=False, needs_layout_passes=False
        ),
    )
    def zw(x_hbm, z_hbm, out_hbm, ib0, ib1, ob0, ob1, si0, si1, so0, so1):
        wid = lax.axis_index("s") * NC + lax.axis_index("c")

        in_bufs, in_sems = (ib0, ib1), (si0, si1)
        out_bufs, out_sems = (ob0, ob1), (so0, so1)

        def in_src(g):
            ci, k = divmod(g, NK)
            return x_hbm.at[wid * ch_per + ci, pl.ds(RI * k, RI), :]

        pltpu.async_copy(in_src(0), ib0, si0)
        pltpu.async_copy(in_src(1), ib1, si1)
        pltpu.async_copy(z_hbm, ob0, so0)
        pltpu.async_copy(z_hbm, ob1, so1)

        iota = lax.iota(jnp.int32, L)
        cvecs = [2 * (k * L + iota) for k in range(W // L)]

        rows_of = lambda g: RO if g % NK != NK - 1 else Ho - RO * (NK - 1)

        for g in range(NG):
            ci, k = divmod(g, NK)
            ch = wid * ch_per + ci
            q = g % 2
            ib, ob = in_bufs[q], out_bufs[q]

            pltpu.make_async_copy(in_src(g), ib, in_sems[q]).wait()
            if g < 2:
                pltpu.make_async_copy(z_hbm, ob, out_sems[q]).wait()
            else:
                gp = g - 2
                chp = wid * ch_per + gp // NK
                pltpu.make_async_copy(
                    ob.at[pl.ds(0, rows_of(gp))],
                    out_hbm.at[chp // NCH, chp % NCH,
                               pl.ds(RO * (gp % NK), rows_of(gp)), :],
                    out_sems[q],
                ).wait()

            def do_row(m, c2):
                rvec = lax.broadcast(2 * m, (L,))
                for kk in range(W // L):
                    plsc.store_scatter(
                        ob, [rvec, cvecs[kk]], ib[m, pl.ds(kk * L, L)]
                    )
                return c2
            lax.fori_loop(0, RI, do_row, 0)

            pltpu.async_copy(
                ob.at[pl.ds(0, rows_of(g))],
                out_hbm.at[ch // NCH, ch % NCH, pl.ds(RO * k, rows_of(g)), :],
                out_sems[q],
            )
            if g + 2 < NG:
                pltpu.async_copy(in_src(g + 2), ib, in_sems[q])

        for gl in (NG - 2, NG - 1):
            q = gl % 2
            chl = wid * ch_per + gl // NK
            pltpu.make_async_copy(
                out_bufs[q].at[pl.ds(0, rows_of(gl))],
                out_hbm.at[chl // NCH, chl % NCH,
                           pl.ds(RO * (gl % NK), rows_of(gl)), :],
                out_sems[q],
            ).wait()

    return zw(x3, ztile)


def kernel(x):
    B, C, H, W = x.shape
    x3 = x.reshape(B * C, H, W)
    ztile = jnp.zeros((RO, 2 * W - 1), jnp.float32)
    return _zero_weave_sc(x3, ztile, BC=B * C, NCH=C, H=H, W=W)
